# baseline (speedup 1.0000x reference)
import functools

import jax
import jax.numpy as jnp
from jax import lax
from jax.experimental import pallas as pl
from jax.experimental.pallas import tpu as pltpu
from jax.experimental.pallas import tpu_sc as plsc

N_BUCKETS = 1000000
B, N, D = 4096, 200, 32
NP = 208
NC, NS = 2, 16
NW = NC * NS
BPW = B // NW
PH = 2
RPP = BPW // PH
L = 16
G0 = 128
G1 = N - G0
CA = G0 // L
CB = NP // L - CA
RB = NP - G0
TROWS = N_BUCKETS * D // 128
NT = N_BUCKETS // 128
NTR = N_BUCKETS - NT * 128


def _pack_body(embt_hbm, out_hbm, in_v, out_v, isem0, isem1, osem0, osem1):
    wid = lax.axis_index("s") * NC + lax.axis_index("c")
    isems = (isem0, isem1)
    osems = (osem0, osem1)
    per = NT // NW
    extra = NT - per * NW
    g0 = wid * per + lax.min(wid, jnp.int32(extra))
    ng = per + jnp.where(wid < extra, 1, 0).astype(jnp.int32)

    lanes = lax.iota(jnp.int32, L)

    def transform(s, nrow):
        def m_step(m, carry):
            i = m >> 3
            q = (m & 7) >> 1
            d0 = (m & 1) * L
            colidx = jnp.full((L,), 1, jnp.int32) * (4 * i + q)
            vals = plsc.load_gather(in_v.at[s], [lanes + d0, colidx])
            out_v[s, i, pl.ds(q * D + d0, L)] = vals
            return carry

        lax.fori_loop(0, nrow * 8, m_step, 0)

    def stage(g, s):
        pltpu.async_copy(embt_hbm.at[:, pl.ds((g0 + g) * 128, 128)],
                         in_v.at[s], isems[s])

    def stage_wait(s):
        pltpu.make_async_copy(embt_hbm.at[:, pl.ds(0, 128)],
                              in_v.at[s], isems[s]).wait()

    def flush(g, s):
        pltpu.async_copy(out_v.at[s], out_hbm.at[pl.ds((g0 + g) * 32, 32)],
                         osems[s])

    def flush_wait(s):
        pltpu.make_async_copy(out_v.at[s], out_hbm.at[pl.ds(0, 32)],
                              osems[s]).wait()

    stage(0, 0)
    stage(1, 1)

    def step(g, carry):
        for s in range(2):
            gg = g * 2 + s

            @pl.when(gg < ng)
            def _():
                stage_wait(s)

                @pl.when(gg >= 2)
                def _():
                    flush_wait(s)
                transform(s, 32)
                flush(gg, s)

                @pl.when(gg + 2 < ng)
                def _():
                    stage(gg + 2, s)
        return carry

    lax.fori_loop(0, (ng + 1) // 2, step, 0)
    flush_wait(0)
    flush_wait(1)

    @pl.when(wid == 0)
    def _():
        for d in range(D):
            pltpu.sync_copy(embt_hbm.at[d, pl.ds(NT * 128, NTR)],
                            in_v.at[0, d, pl.ds(0, NTR)])
        transform(0, NTR // 4)
        pltpu.sync_copy(out_v.at[0, pl.ds(0, NTR // 4)],
                        out_hbm.at[pl.ds(NT * 32, NTR // 4)])


def _fh_body(idx_hbm, val_hbm, emb_hbm, out_hbm,
             idx_v, w_v, q_v, ra_v, rb_v, out_v, sem0, sem1):
    wid = lax.axis_index("s") * NC + lax.axis_index("c")
    sems = (sem0, sem1)

    zero = jnp.zeros((L,), jnp.float32)
    for s in range(2):
        def zrow(i, c):
            for j in range(128 // L):
                rb_v[s, i, pl.ds(j * L, L)] = zero
            return c
        lax.fori_loop(G1, RB, zrow, 0)

    def prep_row(r):
        def chunk(c, carry):
            off = c * L
            x = idx_v[r, pl.ds(off, L)]
            v = w_v[r, pl.ds(off, L)]
            bucket = lax.rem(x, N_BUCKETS)
            idx_v[r, pl.ds(off, L)] = lax.shift_right_logical(bucket, 2)
            q_v[r, pl.ds(off, L)] = (bucket & 3) * D
            w_v[r, pl.ds(off, L)] = (2 * (x & 1) - 1).astype(jnp.float32) * v
            return carry

        lax.fori_loop(0, NP // L, chunk, 0)

    def gather_parts(r, b):
        yield (emb_hbm.at[idx_v.at[r, pl.ds(0, G0)]],
               ra_v.at[b], sems[b])
        yield (emb_hbm.at[idx_v.at[r, pl.ds(G0, G1)]],
               rb_v.at[b, pl.ds(0, G1)], sems[b])

    def issue(r, b):
        for src, dst, sem in gather_parts(r, b):
            pltpu.async_copy(src, dst, sem)

    def wait(r, b):
        for src, dst, sem in gather_parts(r, b):
            pltpu.make_async_copy(src, dst, sem).wait()

    def compute_row(r, b):
        def tree_sum(ps):
            while len(ps) > 1:
                ps = [ps[i] + ps[i + 1] for i in range(0, len(ps) - 1, 2)] + (
                    [ps[-1]] if len(ps) % 2 else [])
            return ps[0]

        def ring_chunks(ring, c0, n_chunks, base, carry):
            def acc_chunk(c, carry):
                a0, a1 = carry
                off = (c0 + c) * L
                wv = w_v[r, pl.ds(off, L)]
                qv = q_v[r, pl.ds(off, L)]
                p0, p1 = [], []
                for k in range(L):
                    w = wv[k]
                    q = qv[k]
                    n = (c0 + c) * L + k - base
                    p0.append(ring[b, n, pl.ds(q, L)] * w)
                    p1.append(ring[b, n, pl.ds(q + L, L)] * w)
                return a0 + tree_sum(p0), a1 + tree_sum(p1)

            return lax.fori_loop(0, n_chunks, acc_chunk, carry)

        acc = (jnp.zeros((L,), jnp.float32), jnp.zeros((L,), jnp.float32))
        acc = ring_chunks(ra_v, 0, CA, 0, acc)
        a0, a1 = ring_chunks(rb_v, CA, CB, G0, acc)
        out_v[r, pl.ds(0, L)] = a0
        out_v[r, pl.ds(L, L)] = a1

    def phase(p, carry):
        base = wid * BPW + p * RPP
        pltpu.sync_copy(idx_hbm.at[pl.ds(base, RPP)], idx_v)
        pltpu.sync_copy(val_hbm.at[pl.ds(base, RPP)], w_v)
        lax.fori_loop(0, RPP, lambda r, c: (prep_row(r), c)[1], 0)

        issue(0, 0)
        issue(1, 1)

        def step(g, c):
            for b in range(2):
                r = g * 2 + b
                wait(r, b)
                compute_row(r, b)

                @pl.when(r + 2 < RPP)
                def _():
                    issue(r + 2, b)
            return c

        lax.fori_loop(0, RPP // 2, step, 0)
        pltpu.sync_copy(out_v, out_hbm.at[pl.ds(base, RPP)])
        return carry

    lax.fori_loop(0, PH, phase, 0)


@jax.jit
def _fh_sc(indices, values, embedding):
    mesh = plsc.VectorSubcoreMesh(core_axis_name="c", subcore_axis_name="s",
                                  num_cores=NC, num_subcores=NS)
    packed = pl.kernel(
        _pack_body,
        out_type=jax.ShapeDtypeStruct((TROWS, 128), jnp.float32),
        mesh=mesh,
        compiler_params=pltpu.CompilerParams(needs_layout_passes=False),
        scratch_types=[
            pltpu.VMEM((2, D, 128), jnp.float32),
            pltpu.VMEM((2, 32, 128), jnp.float32),
            pltpu.SemaphoreType.DMA,
            pltpu.SemaphoreType.DMA,
            pltpu.SemaphoreType.DMA,
            pltpu.SemaphoreType.DMA,
        ],
    )(embedding.T)

    idx_p = jnp.pad(indices, ((0, 0), (0, NP - N)))
    val_p = jnp.pad(values, ((0, 0), (0, NP - N)))
    return pl.kernel(
        _fh_body,
        out_type=jax.ShapeDtypeStruct((B, D), jnp.float32),
        mesh=mesh,
        scratch_types=[
            pltpu.VMEM((RPP, NP), jnp.int32),
            pltpu.VMEM((RPP, NP), jnp.float32),
            pltpu.VMEM((RPP, NP), jnp.int32),
            pltpu.VMEM((2, G0, 128), jnp.float32),
            pltpu.VMEM((2, RB, 128), jnp.float32),
            pltpu.VMEM((RPP, D), jnp.float32),
            pltpu.SemaphoreType.DMA,
            pltpu.SemaphoreType.DMA,
        ],
    )(idx_p, val_p, packed)


def kernel(indices, values, embedding):
    return _fh_sc(indices.astype(jnp.int32), values, embedding)


# speedup vs baseline: 1.0920x; 1.0920x over previous
import functools

import jax
import jax.numpy as jnp
from jax import lax
from jax.experimental import pallas as pl
from jax.experimental.pallas import tpu as pltpu
from jax.experimental.pallas import tpu_sc as plsc

N_BUCKETS = 1000000
B, N, D = 4096, 200, 32
NP = 208
NC, NS = 2, 16
NW = NC * NS
BPW = B // NW
PH = 2
RPP = BPW // PH
L = 16
G0 = 128
G1 = N - G0
CA = G0 // L
CB = NP // L - CA
RB = NP - G0
TROWS = N_BUCKETS * D // 128
NT = N_BUCKETS // 128
NTR = N_BUCKETS - NT * 128


def _pack_body(embt_hbm, out_hbm, in_v, out_v, isem0, isem1, osem0, osem1):
    wid = lax.axis_index("s") * NC + lax.axis_index("c")
    isems = (isem0, isem1)
    osems = (osem0, osem1)
    per = NT // NW
    extra = NT - per * NW
    g0 = wid * per + lax.min(wid, jnp.int32(extra))
    ng = per + jnp.where(wid < extra, 1, 0).astype(jnp.int32)

    lanes = lax.iota(jnp.int32, L)

    lanes_hi = lanes + L
    ones = jnp.full((L,), 1, jnp.int32)

    def transform(s, nrow):
        def i_step(i, carry):
            for q in range(4):
                colidx = ones * (4 * i + q)
                v0 = plsc.load_gather(in_v.at[s], [lanes, colidx])
                v1 = plsc.load_gather(in_v.at[s], [lanes_hi, colidx])
                out_v[s, i, pl.ds(q * D, L)] = v0
                out_v[s, i, pl.ds(q * D + L, L)] = v1
            return carry

        lax.fori_loop(0, nrow, i_step, 0, unroll=4)

    def stage(g, s):
        pltpu.async_copy(embt_hbm.at[:, pl.ds((g0 + g) * 128, 128)],
                         in_v.at[s], isems[s])

    def stage_wait(s):
        pltpu.make_async_copy(embt_hbm.at[:, pl.ds(0, 128)],
                              in_v.at[s], isems[s]).wait()

    def flush(g, s):
        pltpu.async_copy(out_v.at[s], out_hbm.at[pl.ds((g0 + g) * 32, 32)],
                         osems[s])

    def flush_wait(s):
        pltpu.make_async_copy(out_v.at[s], out_hbm.at[pl.ds(0, 32)],
                              osems[s]).wait()

    stage(0, 0)
    stage(1, 1)

    def step(g, carry):
        for s in range(2):
            gg = g * 2 + s

            @pl.when(gg < ng)
            def _():
                stage_wait(s)

                @pl.when(gg >= 2)
                def _():
                    flush_wait(s)
                transform(s, 32)
                flush(gg, s)

                @pl.when(gg + 2 < ng)
                def _():
                    stage(gg + 2, s)
        return carry

    lax.fori_loop(0, (ng + 1) // 2, step, 0)
    flush_wait(0)
    flush_wait(1)

    @pl.when(wid == 0)
    def _():
        for d in range(D):
            pltpu.sync_copy(embt_hbm.at[d, pl.ds(NT * 128, NTR)],
                            in_v.at[0, d, pl.ds(0, NTR)])
        transform(0, NTR // 4)
        pltpu.sync_copy(out_v.at[0, pl.ds(0, NTR // 4)],
                        out_hbm.at[pl.ds(NT * 32, NTR // 4)])


def _fh_body(idx_hbm, val_hbm, emb_hbm, out_hbm,
             idx_v, w_v, q_v, ra_v, rb_v, out_v, sem0, sem1):
    wid = lax.axis_index("s") * NC + lax.axis_index("c")
    sems = (sem0, sem1)

    zero = jnp.zeros((L,), jnp.float32)
    for s in range(2):
        def zrow(i, c):
            for j in range(128 // L):
                rb_v[s, i, pl.ds(j * L, L)] = zero
            return c
        lax.fori_loop(G1, RB, zrow, 0)

    def prep_row(r):
        def chunk(c, carry):
            off = c * L
            x = idx_v[r, pl.ds(off, L)]
            v = w_v[r, pl.ds(off, L)]
            bucket = lax.rem(x, N_BUCKETS)
            idx_v[r, pl.ds(off, L)] = lax.shift_right_logical(bucket, 2)
            q_v[r, pl.ds(off, L)] = (bucket & 3) * D
            w_v[r, pl.ds(off, L)] = (2 * (x & 1) - 1).astype(jnp.float32) * v
            return carry

        lax.fori_loop(0, NP // L, chunk, 0)

    def gather_parts(r, b):
        yield (emb_hbm.at[idx_v.at[r, pl.ds(0, G0)]],
               ra_v.at[b], sems[b])
        yield (emb_hbm.at[idx_v.at[r, pl.ds(G0, G1)]],
               rb_v.at[b, pl.ds(0, G1)], sems[b])

    def issue(r, b):
        for src, dst, sem in gather_parts(r, b):
            pltpu.async_copy(src, dst, sem)

    def wait(r, b):
        for src, dst, sem in gather_parts(r, b):
            pltpu.make_async_copy(src, dst, sem).wait()

    def compute_row(r, b):
        def tree_sum(ps):
            while len(ps) > 1:
                ps = [ps[i] + ps[i + 1] for i in range(0, len(ps) - 1, 2)] + (
                    [ps[-1]] if len(ps) % 2 else [])
            return ps[0]

        def ring_chunks(ring, c0, n_chunks, base, carry):
            def acc_chunk(c, carry):
                a0, a1 = carry
                off = (c0 + c) * L
                wv = w_v[r, pl.ds(off, L)]
                qv = q_v[r, pl.ds(off, L)]
                p0, p1 = [], []
                for k in range(L):
                    w = wv[k]
                    q = qv[k]
                    n = (c0 + c) * L + k - base
                    p0.append(ring[b, n, pl.ds(q, L)] * w)
                    p1.append(ring[b, n, pl.ds(q + L, L)] * w)
                return a0 + tree_sum(p0), a1 + tree_sum(p1)

            return lax.fori_loop(0, n_chunks, acc_chunk, carry)

        acc = (jnp.zeros((L,), jnp.float32), jnp.zeros((L,), jnp.float32))
        acc = ring_chunks(ra_v, 0, CA, 0, acc)
        a0, a1 = ring_chunks(rb_v, CA, CB, G0, acc)
        out_v[r, pl.ds(0, L)] = a0
        out_v[r, pl.ds(L, L)] = a1

    def phase(p, carry):
        base = wid * BPW + p * RPP
        pltpu.sync_copy(idx_hbm.at[pl.ds(base, RPP)], idx_v)
        pltpu.sync_copy(val_hbm.at[pl.ds(base, RPP)], w_v)
        lax.fori_loop(0, RPP, lambda r, c: (prep_row(r), c)[1], 0)

        issue(0, 0)
        issue(1, 1)

        def step(g, c):
            for b in range(2):
                r = g * 2 + b
                wait(r, b)
                compute_row(r, b)

                @pl.when(r + 2 < RPP)
                def _():
                    issue(r + 2, b)
            return c

        lax.fori_loop(0, RPP // 2, step, 0)
        pltpu.sync_copy(out_v, out_hbm.at[pl.ds(base, RPP)])
        return carry

    lax.fori_loop(0, PH, phase, 0)


@jax.jit
def _fh_sc(indices, values, embedding):
    mesh = plsc.VectorSubcoreMesh(core_axis_name="c", subcore_axis_name="s",
                                  num_cores=NC, num_subcores=NS)
    packed = pl.kernel(
        _pack_body,
        out_type=jax.ShapeDtypeStruct((TROWS, 128), jnp.float32),
        mesh=mesh,
        compiler_params=pltpu.CompilerParams(needs_layout_passes=False),
        scratch_types=[
            pltpu.VMEM((2, D, 128), jnp.float32),
            pltpu.VMEM((2, 32, 128), jnp.float32),
            pltpu.SemaphoreType.DMA,
            pltpu.SemaphoreType.DMA,
            pltpu.SemaphoreType.DMA,
            pltpu.SemaphoreType.DMA,
        ],
    )(embedding.T)

    idx_p = jnp.pad(indices, ((0, 0), (0, NP - N)))
    val_p = jnp.pad(values, ((0, 0), (0, NP - N)))
    return pl.kernel(
        _fh_body,
        out_type=jax.ShapeDtypeStruct((B, D), jnp.float32),
        mesh=mesh,
        scratch_types=[
            pltpu.VMEM((RPP, NP), jnp.int32),
            pltpu.VMEM((RPP, NP), jnp.float32),
            pltpu.VMEM((RPP, NP), jnp.int32),
            pltpu.VMEM((2, G0, 128), jnp.float32),
            pltpu.VMEM((2, RB, 128), jnp.float32),
            pltpu.VMEM((RPP, D), jnp.float32),
            pltpu.SemaphoreType.DMA,
            pltpu.SemaphoreType.DMA,
        ],
    )(idx_p, val_p, packed)


def kernel(indices, values, embedding):
    return _fh_sc(indices.astype(jnp.int32), values, embedding)


# speedup vs baseline: 1.0950x; 1.0028x over previous
import functools

import jax
import jax.numpy as jnp
from jax import lax
from jax.experimental import pallas as pl
from jax.experimental.pallas import tpu as pltpu
from jax.experimental.pallas import tpu_sc as plsc

N_BUCKETS = 1000000
B, N, D = 4096, 200, 32
NP = 208
NC, NS = 2, 16
NW = NC * NS
BPW = B // NW
PH = 2
RPP = BPW // PH
L = 16
G0 = 128
G1 = N - G0
CA = G0 // L
CB = NP // L - CA
RB = NP - G0
TROWS = N_BUCKETS * D // 128
NT = N_BUCKETS // 128
NTR = N_BUCKETS - NT * 128


def _pack_body(embt_hbm, out_hbm, in_v, out_v,
               isem0, isem1, isem2, isem3, osem0, osem1, osem2, osem3):
    wid = lax.axis_index("s") * NC + lax.axis_index("c")
    isems = (isem0, isem1, isem2, isem3)
    osems = (osem0, osem1, osem2, osem3)
    per = NT // NW
    extra = NT - per * NW
    g0 = wid * per + lax.min(wid, jnp.int32(extra))
    ng = per + jnp.where(wid < extra, 1, 0).astype(jnp.int32)

    lanes = lax.iota(jnp.int32, L)

    lanes_hi = lanes + L
    ones = jnp.full((L,), 1, jnp.int32)

    def transform(s, nrow):
        def i_step(i, carry):
            for q in range(4):
                colidx = ones * (4 * i + q)
                v0 = plsc.load_gather(in_v.at[s], [lanes, colidx])
                v1 = plsc.load_gather(in_v.at[s], [lanes_hi, colidx])
                out_v[s, i, pl.ds(q * D, L)] = v0
                out_v[s, i, pl.ds(q * D + L, L)] = v1
            return carry

        lax.fori_loop(0, nrow, i_step, 0, unroll=4)

    def stage(g, s):
        pltpu.async_copy(embt_hbm.at[:, pl.ds((g0 + g) * 128, 128)],
                         in_v.at[s], isems[s])

    def stage_wait(s):
        pltpu.make_async_copy(embt_hbm.at[:, pl.ds(0, 128)],
                              in_v.at[s], isems[s]).wait()

    def flush(g, s):
        pltpu.async_copy(out_v.at[s], out_hbm.at[pl.ds((g0 + g) * 32, 32)],
                         osems[s])

    def flush_wait(s):
        pltpu.make_async_copy(out_v.at[s], out_hbm.at[pl.ds(0, 32)],
                              osems[s]).wait()

    NSLOT = 4
    for s in range(NSLOT):
        stage(s, s)

    def step(g, carry):
        for s in range(NSLOT):
            gg = g * NSLOT + s

            @pl.when(gg < ng)
            def _():
                stage_wait(s)

                @pl.when(gg >= NSLOT)
                def _():
                    flush_wait(s)
                transform(s, 32)
                flush(gg, s)

                @pl.when(gg + NSLOT < ng)
                def _():
                    stage(gg + NSLOT, s)
        return carry

    lax.fori_loop(0, (ng + NSLOT - 1) // NSLOT, step, 0)
    for s in range(NSLOT):
        flush_wait(s)

    @pl.when(wid == 0)
    def _():
        for d in range(D):
            pltpu.sync_copy(embt_hbm.at[d, pl.ds(NT * 128, NTR)],
                            in_v.at[0, d, pl.ds(0, NTR)])
        transform(0, NTR // 4)
        pltpu.sync_copy(out_v.at[0, pl.ds(0, NTR // 4)],
                        out_hbm.at[pl.ds(NT * 32, NTR // 4)])


def _fh_body(idx_hbm, val_hbm, emb_hbm, out_hbm,
             idx_v, w_v, q_v, ra_v, rb_v, out_v, sem0, sem1):
    wid = lax.axis_index("s") * NC + lax.axis_index("c")
    sems = (sem0, sem1)

    zero = jnp.zeros((L,), jnp.float32)
    for s in range(2):
        def zrow(i, c):
            for j in range(128 // L):
                rb_v[s, i, pl.ds(j * L, L)] = zero
            return c
        lax.fori_loop(G1, RB, zrow, 0)

    def prep_row(r):
        def chunk(c, carry):
            off = c * L
            x = idx_v[r, pl.ds(off, L)]
            v = w_v[r, pl.ds(off, L)]
            bucket = lax.rem(x, N_BUCKETS)
            idx_v[r, pl.ds(off, L)] = lax.shift_right_logical(bucket, 2)
            q_v[r, pl.ds(off, L)] = (bucket & 3) * D
            w_v[r, pl.ds(off, L)] = (2 * (x & 1) - 1).astype(jnp.float32) * v
            return carry

        lax.fori_loop(0, NP // L, chunk, 0)

    def gather_parts(r, b):
        yield (emb_hbm.at[idx_v.at[r, pl.ds(0, G0)]],
               ra_v.at[b], sems[b])
        yield (emb_hbm.at[idx_v.at[r, pl.ds(G0, G1)]],
               rb_v.at[b, pl.ds(0, G1)], sems[b])

    def issue(r, b):
        for src, dst, sem in gather_parts(r, b):
            pltpu.async_copy(src, dst, sem)

    def wait(r, b):
        for src, dst, sem in gather_parts(r, b):
            pltpu.make_async_copy(src, dst, sem).wait()

    def compute_row(r, b):
        def tree_sum(ps):
            while len(ps) > 1:
                ps = [ps[i] + ps[i + 1] for i in range(0, len(ps) - 1, 2)] + (
                    [ps[-1]] if len(ps) % 2 else [])
            return ps[0]

        def ring_chunks(ring, c0, n_chunks, base, carry):
            def acc_chunk(c, carry):
                a0, a1 = carry
                off = (c0 + c) * L
                wv = w_v[r, pl.ds(off, L)]
                qv = q_v[r, pl.ds(off, L)]
                p0, p1 = [], []
                for k in range(L):
                    w = wv[k]
                    q = qv[k]
                    n = (c0 + c) * L + k - base
                    p0.append(ring[b, n, pl.ds(q, L)] * w)
                    p1.append(ring[b, n, pl.ds(q + L, L)] * w)
                return a0 + tree_sum(p0), a1 + tree_sum(p1)

            return lax.fori_loop(0, n_chunks, acc_chunk, carry)

        acc = (jnp.zeros((L,), jnp.float32), jnp.zeros((L,), jnp.float32))
        acc = ring_chunks(ra_v, 0, CA, 0, acc)
        a0, a1 = ring_chunks(rb_v, CA, CB, G0, acc)
        out_v[r, pl.ds(0, L)] = a0
        out_v[r, pl.ds(L, L)] = a1

    def phase(p, carry):
        base = wid * BPW + p * RPP
        pltpu.sync_copy(idx_hbm.at[pl.ds(base, RPP)], idx_v)
        pltpu.sync_copy(val_hbm.at[pl.ds(base, RPP)], w_v)
        lax.fori_loop(0, RPP, lambda r, c: (prep_row(r), c)[1], 0)

        issue(0, 0)
        issue(1, 1)

        def step(g, c):
            for b in range(2):
                r = g * 2 + b
                wait(r, b)
                compute_row(r, b)

                @pl.when(r + 2 < RPP)
                def _():
                    issue(r + 2, b)
            return c

        lax.fori_loop(0, RPP // 2, step, 0)
        pltpu.sync_copy(out_v, out_hbm.at[pl.ds(base, RPP)])
        return carry

    lax.fori_loop(0, PH, phase, 0)


@jax.jit
def _fh_sc(indices, values, embedding):
    mesh = plsc.VectorSubcoreMesh(core_axis_name="c", subcore_axis_name="s",
                                  num_cores=NC, num_subcores=NS)
    packed = pl.kernel(
        _pack_body,
        out_type=jax.ShapeDtypeStruct((TROWS, 128), jnp.float32),
        mesh=mesh,
        compiler_params=pltpu.CompilerParams(needs_layout_passes=False),
        scratch_types=[
            pltpu.VMEM((4, D, 128), jnp.float32),
            pltpu.VMEM((4, 32, 128), jnp.float32),
        ] + [pltpu.SemaphoreType.DMA] * 8,
    )(embedding.T)

    idx_p = jnp.pad(indices, ((0, 0), (0, NP - N)))
    val_p = jnp.pad(values, ((0, 0), (0, NP - N)))
    return pl.kernel(
        _fh_body,
        out_type=jax.ShapeDtypeStruct((B, D), jnp.float32),
        mesh=mesh,
        scratch_types=[
            pltpu.VMEM((RPP, NP), jnp.int32),
            pltpu.VMEM((RPP, NP), jnp.float32),
            pltpu.VMEM((RPP, NP), jnp.int32),
            pltpu.VMEM((2, G0, 128), jnp.float32),
            pltpu.VMEM((2, RB, 128), jnp.float32),
            pltpu.VMEM((RPP, D), jnp.float32),
            pltpu.SemaphoreType.DMA,
            pltpu.SemaphoreType.DMA,
        ],
    )(idx_p, val_p, packed)


def kernel(indices, values, embedding):
    return _fh_sc(indices.astype(jnp.int32), values, embedding)


# speedup vs baseline: 1.7118x; 1.5632x over previous
import functools

import jax
import jax.numpy as jnp
from jax import lax
from jax.experimental import pallas as pl
from jax.experimental.pallas import tpu as pltpu
from jax.experimental.pallas import tpu_sc as plsc

N_BUCKETS = 1000000
B, N, D = 4096, 200, 32
NC, NS = 2, 16
NW = NC * NS
BPW = B // NW
L = 16
NFULL = N // L
TAIL = N - NFULL * L
TAIL_OFF = N - L
K = 8
G0 = 128
G1 = N - G0


def _sc_body(idx_hbm, val_hbm, emb_hbm, out_hbm, idx_v, w_v, rows_v, out_v, *sems):
    wid = lax.axis_index("s") * NC + lax.axis_index("c")
    base = wid * BPW

    pltpu.sync_copy(idx_hbm.at[pl.ds(base, BPW)], idx_v)
    pltpu.sync_copy(val_hbm.at[pl.ds(base, BPW)], w_v)

    def prep_row(r):
        def chunk(j, carry):
            off = j * L
            x = idx_v[r, pl.ds(off, L)]
            v = w_v[r, pl.ds(off, L)]
            idx_v[r, pl.ds(off, L)] = lax.rem(x, N_BUCKETS)
            w_v[r, pl.ds(off, L)] = (2 * (x & 1) - 1).astype(jnp.float32) * v
            return carry

        lax.fori_loop(0, NFULL, chunk, 0)
        x = idx_v[r, pl.ds(TAIL_OFF, L)]
        v = w_v[r, pl.ds(TAIL_OFF, L)]
        s = (2 * (x & 1) - 1).astype(jnp.float32)
        lane = lax.iota(jnp.int32, L)
        idx_v[r, pl.ds(TAIL_OFF, L)] = lax.rem(x, N_BUCKETS)
        w_v[r, pl.ds(TAIL_OFF, L)] = jnp.where(lane < (L - TAIL), v, s * v)

    def gather_parts(r, b):
        yield (emb_hbm.at[idx_v.at[r, pl.ds(0, G0)]],
               rows_v.at[b, pl.ds(0, G0)], sems[b])
        yield (emb_hbm.at[idx_v.at[r, pl.ds(G0, G1)]],
               rows_v.at[b, pl.ds(G0, G1)], sems[b])

    def issue(r, b):
        for src, dst, sem in gather_parts(r, b):
            pltpu.async_copy(src, dst, sem)

    def wait(r, b):
        for src, dst, sem in gather_parts(r, b):
            pltpu.make_async_copy(src, dst, sem).wait()

    def compute_row(r, b):
        def tree_sum(ps):
            while len(ps) > 1:
                ps = [ps[i] + ps[i + 1] for i in range(0, len(ps) - 1, 2)] + (
                    [ps[-1]] if len(ps) % 2 else [])
            return ps[0]

        def acc_chunk(c, carry):
            a0, a1 = carry
            n0 = c * L
            wv = w_v[r, pl.ds(n0, L)]
            p0 = [rows_v[b, n0 + k, pl.ds(0, L)] * wv[k] for k in range(L)]
            p1 = [rows_v[b, n0 + k, pl.ds(L, L)] * wv[k] for k in range(L)]
            return a0 + tree_sum(p0), a1 + tree_sum(p1)

        a0, a1 = lax.fori_loop(
            0, NFULL, acc_chunk,
            (jnp.zeros((L,), jnp.float32), jnp.zeros((L,), jnp.float32)))
        wv = w_v[r, pl.ds(TAIL_OFF, L)]
        p0 = [rows_v[b, TAIL_OFF + k, pl.ds(0, L)] * wv[k]
              for k in range(L - TAIL, L)]
        p1 = [rows_v[b, TAIL_OFF + k, pl.ds(L, L)] * wv[k]
              for k in range(L - TAIL, L)]
        out_v[r, pl.ds(0, L)] = a0 + tree_sum(p0)
        out_v[r, pl.ds(L, L)] = a1 + tree_sum(p1)

    lax.fori_loop(0, BPW, lambda r, c: (prep_row(r), c)[1], 0)

    for b in range(K):
        issue(b, b)

    def outer(g, carry):
        r0 = g * K
        for b in range(K):
            r = r0 + b
            wait(r, b)
            compute_row(r, b)
            nxt = r + K

            @pl.when(nxt < BPW)
            def _():
                issue(nxt, b)
        return carry

    lax.fori_loop(0, BPW // K, outer, 0)

    pltpu.sync_copy(out_v, out_hbm.at[pl.ds(base, BPW)])


@jax.jit
def _fh_sc(indices, values, embedding):
    mesh = plsc.VectorSubcoreMesh(core_axis_name="c", subcore_axis_name="s",
                                  num_cores=NC, num_subcores=NS)
    return pl.kernel(
        _sc_body,
        out_type=jax.ShapeDtypeStruct((B, D), jnp.float32),
        mesh=mesh,
        compiler_params=pltpu.CompilerParams(use_tc_tiling_on_sc=False),
        scratch_types=[
            pltpu.VMEM((BPW, N), jnp.int32),
            pltpu.VMEM((BPW, N), jnp.float32),
            pltpu.VMEM((K, N, D), jnp.float32),
            pltpu.VMEM((BPW, D), jnp.float32),
        ] + [pltpu.SemaphoreType.DMA] * K,
    )(indices, values, embedding)


def kernel(indices, values, embedding):
    return _fh_sc(indices.astype(jnp.int32), values, embedding)
